# Initial kernel scaffold; baseline (speedup 1.0000x reference)
import jax
import jax.numpy as jnp
from jax.experimental import pallas as pl


def kernel(inputs, embedding):
    raise NotImplementedError("write your pallas kernel here")


# speedup vs baseline: 1.5604x; 1.5604x over previous
import functools

import jax
import jax.numpy as jnp
from jax import lax
from jax.experimental import pallas as pl
from jax.experimental.pallas import tpu as pltpu
from jax.experimental.pallas import tpu_sc as plsc

_FEATURES = 32


@functools.lru_cache(maxsize=None)
def _make_lookup(B, D, n_workers, chunk):
    b_per_w = B // n_workers
    n_chunks = b_per_w // chunk
    mesh = plsc.VectorSubcoreMesh(core_axis_name="c", subcore_axis_name="s")

    @functools.partial(
        pl.kernel,
        mesh=mesh,
        out_type=jax.ShapeDtypeStruct((B, D), jnp.float32),
        scratch_types=[
            pltpu.VMEM((chunk,), jnp.int32),
            pltpu.VMEM((chunk, D), jnp.float32),
            pltpu.SemaphoreType.DMA,
        ],
        compiler_params=pltpu.CompilerParams(use_tc_tiling_on_sc=False),
    )
    def lookup(idx_hbm, table_hbm, out_hbm, idx_v, rows_v, sem):
        wid = lax.axis_index("s") * 2 + lax.axis_index("c")
        base = wid * b_per_w

        def body(i, carry):
            off = base + i * chunk
            pltpu.sync_copy(idx_hbm.at[pl.ds(off, chunk)], idx_v)
            pltpu.async_copy(table_hbm.at[idx_v], rows_v, sem).wait()
            pltpu.sync_copy(rows_v, out_hbm.at[pl.ds(off, chunk)])
            return carry

        lax.fori_loop(0, n_chunks, body, 0)

    return lookup


def kernel(inputs, embedding):
    B = inputs.shape[0] * inputs.shape[1]
    flat_idx = inputs.reshape(B).astype(jnp.int32)
    out = _make_lookup(B, _FEATURES, 32, 1664)(flat_idx, embedding)
    return out.reshape(inputs.shape + (_FEATURES,))


# speedup vs baseline: 1.5698x; 1.0060x over previous
import functools

import jax
import jax.numpy as jnp
from jax import lax
from jax.experimental import pallas as pl
from jax.experimental.pallas import tpu as pltpu
from jax.experimental.pallas import tpu_sc as plsc

_FEATURES = 32


@functools.lru_cache(maxsize=None)
def _make_lookup(B, D, n_workers, chunk):
    b_per_w = B // n_workers
    n_chunks = b_per_w // chunk
    mesh = plsc.VectorSubcoreMesh(core_axis_name="c", subcore_axis_name="s")

    @functools.partial(
        pl.kernel,
        mesh=mesh,
        out_type=jax.ShapeDtypeStruct((B, D), jnp.float32),
        scratch_types=[
            pltpu.VMEM((2, chunk), jnp.int32),
            pltpu.VMEM((2, chunk, D), jnp.float32),
            pltpu.SemaphoreType.DMA,
            pltpu.SemaphoreType.DMA,
        ],
        compiler_params=pltpu.CompilerParams(use_tc_tiling_on_sc=False),
    )
    def lookup(idx_hbm, table_hbm, out_hbm, idx_v, rows_v, sem_g, sem_o):
        wid = lax.axis_index("s") * 2 + lax.axis_index("c")
        base = wid * b_per_w

        pltpu.sync_copy(idx_hbm.at[pl.ds(base, chunk)], idx_v.at[0])
        gathers = [
            pltpu.async_copy(table_hbm.at[idx_v.at[0]], rows_v.at[0], sem_g)
        ]
        outs = [None, None]
        for g in range(n_chunks):
            s = g % 2
            ns = (g + 1) % 2
            if g + 1 < n_chunks:
                pltpu.sync_copy(
                    idx_hbm.at[pl.ds(base + (g + 1) * chunk, chunk)],
                    idx_v.at[ns],
                )
            gathers[g].wait()
            if g + 1 < n_chunks:
                if outs[ns] is not None:
                    outs[ns].wait()
                gathers.append(
                    pltpu.async_copy(
                        table_hbm.at[idx_v.at[ns]], rows_v.at[ns], sem_g
                    )
                )
            outs[s] = pltpu.async_copy(
                rows_v.at[s], out_hbm.at[pl.ds(base + g * chunk, chunk)], sem_o
            )
        if n_chunks >= 2:
            outs[(n_chunks - 2) % 2].wait()
        outs[(n_chunks - 1) % 2].wait()

    return lookup


def kernel(inputs, embedding):
    B = inputs.shape[0] * inputs.shape[1]
    flat_idx = inputs.reshape(B).astype(jnp.int32)
    out = _make_lookup(B, _FEATURES, 32, 1664)(flat_idx, embedding)
    return out.reshape(inputs.shape + (_FEATURES,))
